# Initial kernel scaffold; baseline (speedup 1.0000x reference)
import jax
import jax.numpy as jnp
from jax.experimental import pallas as pl


def kernel(x, edge_index, W1, att_src1, att_dst1, b1, W2, att_src2, att_dst2, b2):
    raise NotImplementedError("write your pallas kernel here")


# speedup vs baseline: 9.0212x; 9.0212x over previous
import functools

import jax
import jax.numpy as jnp
from jax import lax
from jax.experimental import pallas as pl
from jax.experimental.pallas import tpu as pltpu
from jax.experimental.pallas import tpu_sc as plsc

N = 10000
E = 320000
DIN = 128
HEADS = 4
CPH = 32
CH = 16
CHB = 400
NWORK = 32
NDW = 10
EPWD = E // NDW
EPWN = E // 16
NPAD = 10240
HALF = 5120
ACCR = 5248
ROWS_PT = ACCR // 16
AROWS = N * 8 // 128
DROWS = NPAD * 4 // 128

_BLK = 200
_GRID = N // _BLK
_BLKC = 80
_GRIDC = N // _BLKC
_CB = HALF // _BLKC

_SC_PARAMS = pltpu.CompilerParams(needs_layout_passes=False)


def _k1_body(x_ref, w_ref, m_ref, h_ref, a_ref):
    h = jnp.dot(x_ref[...], w_ref[...], preferred_element_type=jnp.float32)
    h_ref[...] = h
    a_ref[...] = jnp.dot(h, m_ref[...], preferred_element_type=jnp.float32)


def _k1(x, W, M):
    return pl.pallas_call(
        _k1_body,
        grid=(_GRID,),
        in_specs=[
            pl.BlockSpec((_BLK, DIN), lambda i: (i, 0)),
            pl.BlockSpec((DIN, 128), lambda i: (0, 0)),
            pl.BlockSpec((128, 8), lambda i: (0, 0)),
        ],
        out_specs=[
            pl.BlockSpec((_BLK, 128), lambda i: (i, 0)),
            pl.BlockSpec((_BLK, 8), lambda i: (i, 0)),
        ],
        out_shape=[
            jax.ShapeDtypeStruct((N, 128), jnp.float32),
            jax.ShapeDtypeStruct((N, 8), jnp.float32),
        ],
    )(x, W, M)


def _k2_body(a_ref, s4_ref, amx_ref, sw_ref):
    a_s = a_ref[:, 0:4]
    a_d = a_ref[:, 4:8]
    am = jnp.max(a_s, axis=0, keepdims=True)
    amx_ref[...] = jnp.dot(am, s4_ref[...], preferred_element_type=jnp.float32)
    t = am + a_d
    mh = jnp.where(t >= 0, t, 0.2 * t)
    t2 = a_s + a_d
    e2 = jnp.where(t2 >= 0, t2, 0.2 * t2) - mh
    sw_ref[...] = jnp.exp(e2)


def _k2(A, S4):
    return pl.pallas_call(
        _k2_body,
        grid=(1,),
        in_specs=[
            pl.BlockSpec((N, 8), lambda i: (0, 0)),
            pl.BlockSpec((4, 128), lambda i: (0, 0)),
        ],
        out_specs=[
            pl.BlockSpec((1, 128), lambda i: (0, 0)),
            pl.BlockSpec((N, 4), lambda i: (0, 0)),
        ],
        out_shape=[
            jax.ShapeDtypeStruct((1, 128), jnp.float32),
            jax.ShapeDtypeStruct((N, 4), jnp.float32),
        ],
    )(A, S4)


def _kc_body(p_ref, d_ref, h_ref, sw_ref, b_ref, r_ref, f_ref, out_ref):
    rm = r_ref[...]
    sw = sw_ref[...]
    num = p_ref[0] + jnp.dot(sw, rm, preferred_element_type=jnp.float32) * h_ref[...]
    d4 = jnp.sum(d_ref[...], axis=0) + sw
    den = jnp.dot(d4, rm, preferred_element_type=jnp.float32) + 1e-16
    g = num / den + b_ref[...]
    elu = jnp.where(g > 0, g, jnp.exp(g) - 1.0)
    out_ref[...] = jnp.where(f_ref[...] > 0, elu, g)


def _kc(P, D, h, sw, b, R, flag):
    return pl.pallas_call(
        _kc_body,
        grid=(_GRIDC,),
        in_specs=[
            pl.BlockSpec((1, _BLKC, 128), lambda i: (i // _CB, i % _CB, 0)),
            pl.BlockSpec((NDW, _BLKC, 4), lambda i: (0, i, 0)),
            pl.BlockSpec((_BLKC, 128), lambda i: (i, 0)),
            pl.BlockSpec((_BLKC, 4), lambda i: (i, 0)),
            pl.BlockSpec((1, 128), lambda i: (0, 0)),
            pl.BlockSpec((4, 128), lambda i: (0, 0)),
            pl.BlockSpec((1, 128), lambda i: (0, 0)),
        ],
        out_specs=pl.BlockSpec((_BLKC, 128), lambda i: (i, 0)),
        out_shape=jax.ShapeDtypeStruct((N, 128), jnp.float32),
    )(P, D, h, sw, b, R, flag)


_mesh = plsc.VectorSubcoreMesh(core_axis_name="c", subcore_axis_name="s")


def _edge_weights(atab, amxv, srcv, dstv):
    ws = []
    for h in range(HEADS):
        sflat = lax.shift_left(srcv, 3) + h
        dflat = lax.shift_left(dstv, 3) + (4 + h)
        asv = plsc.load_gather(
            atab, [lax.shift_right_logical(sflat, 7), lax.bitwise_and(sflat, 127)])
        adv = plsc.load_gather(
            atab, [lax.shift_right_logical(dflat, 7), lax.bitwise_and(dflat, 127)])
        t = asv + adv
        e = jnp.where(t >= 0, t, 0.2 * t)
        tm = amxv[h] + adv
        mh = jnp.where(tm >= 0, tm, 0.2 * tm)
        ws.append(jnp.exp(e - mh))
    return ws


@functools.partial(
    pl.kernel,
    mesh=_mesh,
    compiler_params=_SC_PARAMS,
    out_type=jax.ShapeDtypeStruct((2, ACCR, 128), jnp.float32),
    scratch_types=[
        pltpu.VMEM((CH,), jnp.int32),
        pltpu.VMEM((CH,), jnp.int32),
        pltpu.VMEM((CH,), jnp.int32),
        pltpu.VMEM((CH, 128), jnp.float32),
        pltpu.VMEM((CH, 16), jnp.float32),
        pltpu.VMEM((AROWS, 128), jnp.float32),
        pltpu.VMEM((1, 128), jnp.float32),
        pltpu.VMEM_SHARED((ACCR, 128), jnp.float32),
        pltpu.SemaphoreType.DMA,
    ],
)
def _sc_num(pk_hbm, t_hbm, a_hbm, amx_hbm, out_hbm,
            pbuf, sidx, dloc, gsrc, wbuf, atab, amxb, acc, sem):
    cid = lax.axis_index("c")
    sid = lax.axis_index("s")
    iota = lax.iota(jnp.int32, 16)
    zv = jnp.zeros((16,), jnp.float32)
    nbase = cid * HALF

    pltpu.sync_copy(a_hbm, atab)
    pltpu.sync_copy(amx_hbm, amxb)

    def zbody(k, carry):
        for j in range(8):
            gsrc[k, pl.ds(j * 16, 16)] = zv
        return carry

    lax.fori_loop(0, CH, zbody, 0)
    for r in range(ROWS_PT // CH):
        pltpu.sync_copy(gsrc, acc.at[pl.ds(sid * ROWS_PT + r * CH, CH)])
    pltpu.sync_copy(gsrc.at[pl.ds(0, ROWS_PT % CH)],
                    acc.at[pl.ds(sid * ROWS_PT + (ROWS_PT // CH) * CH,
                                 ROWS_PT % CH)])
    plsc.subcore_barrier()

    amxv = amxb[0, pl.ds(0, 16)]

    def chunk(i, carry):
        base = sid * EPWN + i * CH
        pltpu.sync_copy(pk_hbm.at[pl.ds(base, CH)], pbuf)

        for g in range(CH // 16):
            pkv = pbuf[pl.ds(g * 16, 16)]
            srcv = lax.shift_right_logical(pkv, 14)
            dstv = lax.bitwise_and(pkv, 16383)
            ks = iota + g * 16
            plsc.store_scatter(sidx, [ks], srcv)
            loc = dstv - nbase
            inr = (loc >= 0) & (loc < HALF)
            loc = jnp.where(inr, loc, HALF + lax.bitwise_and(ks, 127))
            plsc.store_scatter(dloc, [ks], loc)
            ws = _edge_weights(atab, amxv, srcv, dstv)
            for h in range(HEADS):
                plsc.store_scatter(
                    wbuf, [ks, jnp.full((16,), h, jnp.int32)], ws[h])

        pltpu.async_copy(t_hbm.at[sidx], gsrc, sem).wait()

        def ebody(k, carry2):
            wrow = wbuf[k, pl.ds(0, 16)]
            for h in range(HEADS):
                for v in range(CPH // 16):
                    col = h * CPH + v * 16
                    gsrc[k, pl.ds(col, 16)] = wrow[h] * gsrc[k, pl.ds(col, 16)]
            return carry2

        lax.fori_loop(0, CH, ebody, 0, unroll=4)

        pltpu.sync_copy(gsrc, acc.at[dloc], add=True)
        return carry

    lax.fori_loop(0, EPWN // CH, chunk, 0)
    plsc.subcore_barrier()

    for r in range(ROWS_PT // CH):
        pltpu.sync_copy(acc.at[pl.ds(sid * ROWS_PT + r * CH, CH)], gsrc)
        pltpu.sync_copy(gsrc, out_hbm.at[cid, pl.ds(sid * ROWS_PT + r * CH, CH)])
    pltpu.sync_copy(acc.at[pl.ds(sid * ROWS_PT + (ROWS_PT // CH) * CH,
                                 ROWS_PT % CH)],
                    gsrc.at[pl.ds(0, ROWS_PT % CH)])
    pltpu.sync_copy(gsrc.at[pl.ds(0, ROWS_PT % CH)],
                    out_hbm.at[cid, pl.ds(sid * ROWS_PT + (ROWS_PT // CH) * CH,
                                          ROWS_PT % CH)])


@functools.partial(
    pl.kernel,
    mesh=_mesh,
    compiler_params=_SC_PARAMS,
    out_type=jax.ShapeDtypeStruct((NDW, DROWS, 128), jnp.float32),
    scratch_types=[
        pltpu.VMEM((CHB,), jnp.int32),
        pltpu.VMEM((DROWS, 128), jnp.float32),
        pltpu.VMEM((AROWS, 128), jnp.float32),
        pltpu.VMEM((1, 128), jnp.float32),
    ],
)
def _sc_den(pk_hbm, a_hbm, amx_hbm, out_hbm,
            pbuf, den, atab, amxb):
    cid = lax.axis_index("c")
    sid = lax.axis_index("s")
    wid = cid * (NDW // 2) + sid
    zv = jnp.zeros((16,), jnp.float32)

    pltpu.sync_copy(a_hbm, atab)
    pltpu.sync_copy(amx_hbm, amxb)

    def zbody(r, carry):
        for j in range(8):
            den[r, pl.ds(j * 16, 16)] = zv
        return carry

    lax.fori_loop(0, DROWS, zbody, 0)
    amxv = amxb[0, pl.ds(0, 16)]

    def chunk(i, carry):
        base = wid * EPWD + i * CHB
        pltpu.sync_copy(pk_hbm.at[pl.ds(base, CHB)], pbuf)

        def group(g, carry2):
            pkv = pbuf[pl.ds(g * 16, 16)]
            srcv = lax.shift_right_logical(pkv, 14)
            dstv = lax.bitwise_and(pkv, 16383)
            ws = _edge_weights(atab, amxv, srcv, dstv)
            for h in range(HEADS):
                dflat = lax.shift_left(dstv, 2) + h
                plsc.addupdate_scatter(
                    den,
                    [lax.shift_right_logical(dflat, 7),
                     lax.bitwise_and(dflat, 127)],
                    ws[h])
            return carry2

        lax.fori_loop(0, CHB // 16, group, 0, unroll=2)
        return carry

    @pl.when(sid < NDW // 2)
    def _run():
        lax.fori_loop(0, EPWD // CHB, chunk, 0)
        pltpu.sync_copy(den, out_hbm.at[wid])


def _att_matrix(att_src, att_dst):
    hh = att_src.shape[1]
    cc = att_src.shape[2]
    if hh == 1:
        onehot = jnp.ones((128, 4), jnp.float32)
    else:
        idx_h = (jnp.arange(128, dtype=jnp.int32) * hh) // 128
        onehot = (idx_h[:, None] == jnp.arange(4, dtype=jnp.int32)[None, :]
                  ).astype(jnp.float32)
    msrc = att_src.reshape(hh * cc)[:, None] * onehot
    mdst = att_dst.reshape(hh * cc)[:, None] * onehot
    return jnp.concatenate([msrc, mdst], axis=1)


def _den_sum_view(D):
    return D.reshape(NDW, DROWS * 128)[:, :N * 4].reshape(NDW, N, 4)


def kernel(x, edge_index, W1, att_src1, att_dst1, b1, W2, att_src2, att_dst2, b2):
    epk = edge_index[0] * 16384 + edge_index[1]
    M1 = _att_matrix(att_src1, att_dst1)
    M2 = _att_matrix(att_src2, att_dst2)
    idx_h = (jnp.arange(128, dtype=jnp.int32) * 4) // 128
    R = (jnp.arange(4, dtype=jnp.int32)[:, None] == idx_h[None, :]).astype(jnp.float32)
    S4 = (jnp.arange(4, dtype=jnp.int32)[:, None]
          == jnp.arange(128, dtype=jnp.int32)[None, :]).astype(jnp.float32)
    Wst = jnp.stack([W1, W2])
    Mst = jnp.stack([M1, M2])
    bst = jnp.stack([b1.reshape(1, 128), b2.reshape(1, 128)])
    flagst = jnp.stack([jnp.ones((1, 128), jnp.float32),
                        jnp.zeros((1, 128), jnp.float32)])

    def step(hc, xs):
        W, M, b, flag = xs
        h, A = _k1(hc, W, M)
        amx, sw = _k2(A, S4)
        At = A.reshape(AROWS, 128)
        P = _sc_num(epk, h, At, amx)
        amx_dep = jnp.where(jnp.isnan(P[0, 0:1, :]), 0.0, amx)
        D = _sc_den(epk, At, amx_dep)
        g = _kc(P, _den_sum_view(D), h, sw, b, R, flag)
        return g, 0

    out, _ = lax.scan(step, x, (Wst, Mst, bst, flagst))
    return out
